# baseline (speedup 1.0000x reference)
import functools

import jax
import jax.numpy as jnp
from jax import lax
from jax.experimental import pallas as pl
from jax.experimental.pallas import tpu as pltpu
from jax.experimental.pallas import tpu_sc as plsc

VOCAB = 1000
B = 1024
T = 50

NC = 2
NS = 16
NW = NC * NS
PER_W = (B * T) // NW
K = 64
NCHUNK = PER_W // K


def _make_gather():
  mesh = plsc.VectorSubcoreMesh(core_axis_name="c", subcore_axis_name="s")

  @functools.partial(
      pl.kernel,
      mesh=mesh,
      compiler_params=pltpu.CompilerParams(use_tc_tiling_on_sc=False),
      out_type=jax.ShapeDtypeStruct((B * T, VOCAB), jnp.float32),
      scratch_types=[
          pltpu.VMEM((NCHUNK, K), jnp.int32),
          pltpu.VMEM((2, K, VOCAB), jnp.float32),
          pltpu.SemaphoreType.DMA,
          pltpu.SemaphoreType.DMA,
      ],
  )
  def gather_kernel(table_hbm, idx_hbm, out_hbm, idx_v, buf, gsem, ssem):
    wid = lax.axis_index("s") * NC + lax.axis_index("c")
    base = wid * PER_W
    pltpu.sync_copy(idx_hbm.at[wid], idx_v)
    pltpu.async_copy(table_hbm.at[idx_v.at[0]], buf.at[0], gsem)

    def body(g, carry):
      p = lax.rem(g, 2)
      q = 1 - p
      pltpu.make_async_copy(
          table_hbm.at[idx_v.at[g]], buf.at[p], gsem).wait()

      @pl.when(g >= 1)
      def _():
        pltpu.make_async_copy(
            buf.at[q], out_hbm.at[pl.ds(base, K)], ssem).wait()

      @pl.when(g + 1 < NCHUNK)
      def _():
        pltpu.async_copy(table_hbm.at[idx_v.at[g + 1]], buf.at[q], gsem)

      pltpu.async_copy(buf.at[p], out_hbm.at[pl.ds(base + g * K, K)], ssem)
      return carry

    lax.fori_loop(0, NCHUNK, body, 0)
    pltpu.make_async_copy(
        buf.at[(NCHUNK - 1) % 2], out_hbm.at[pl.ds(base, K)], ssem).wait()

  return gather_kernel


_gather = jax.jit(_make_gather())


def kernel(idx, table):
  idx_chunks = idx.reshape(NW, NCHUNK, K)
  out = _gather(table, idx_chunks)
  return out.reshape(B, T, VOCAB)
